# Initial kernel scaffold; baseline (speedup 1.0000x reference)
import jax
import jax.numpy as jnp
from jax.experimental import pallas as pl


def kernel(nodes, history_uv, history_ra, history_re, feat_table, e_table, r_table, re_table, W_gv, b_gv, W1, b1):
    raise NotImplementedError("write your pallas kernel here")


# speedup vs baseline: 12.0788x; 12.0788x over previous
import functools

import jax
import jax.numpy as jnp
from jax import lax
from jax.experimental import pallas as pl
from jax.experimental.pallas import tpu as pltpu
from jax.experimental.pallas import tpu_sc as plsc

B = 16384
L = 50
V = 100000
R = 5
D = 64

NC = 2
NS = 16
NW = NC * NS
BW = B // NW
PAIR = 2
PROWS = PAIR * L
PPAD = 104
NPAIR_W = BW // PAIR
NBUF = 2

_f32 = jnp.float32


def _stage1_body(e_ref, wt_ref, rr_ref, wb_ref, bg_ref, y_ref, c_ref):
    y_ref[...] = jnp.dot(e_ref[...], wt_ref[...],
                         preferred_element_type=_f32)

    @pl.when(pl.program_id(0) == 0)
    def _():
        c_ref[...] = jnp.dot(rr_ref[...], wb_ref[...],
                             preferred_element_type=_f32) + bg_ref[...]


def _stage1(e_table, wt, rr, wb, bg):
    rows = 800
    grid = V // rows
    return pl.pallas_call(
        _stage1_body,
        grid=(grid,),
        in_specs=[
            pl.BlockSpec((rows, D), lambda i: (i, 0)),
            pl.BlockSpec((D, D), lambda i: (0, 0)),
            pl.BlockSpec((32, D), lambda i: (0, 0)),
            pl.BlockSpec((D, D), lambda i: (0, 0)),
            pl.BlockSpec((1, D), lambda i: (0, 0)),
        ],
        out_specs=[
            pl.BlockSpec((rows, D), lambda i: (i, 0)),
            pl.BlockSpec((32, D), lambda i: (0, 0)),
        ],
        out_shape=[
            jax.ShapeDtypeStruct((V, D), _f32),
            jax.ShapeDtypeStruct((32, D), _f32),
        ],
    )(e_table, wt, rr, wb, bg)


def _stage2_body(y_hbm, uvp_hbm, cidx_hbm, nodes_hbm, feat_hbm, c_hbm,
                 neigh_hbm, selff_hbm,
                 uvp_v, cidx_v, c_v, nodes_v, out_v, rows_v, sem0, sem1):
    wid = lax.axis_index("s") * NC + lax.axis_index("c")
    base = wid * BW
    sems = (sem0, sem1)

    pltpu.sync_copy(c_hbm, c_v)
    pltpu.sync_copy(nodes_hbm.at[pl.ds(base, BW)], nodes_v)
    pltpu.sync_copy(uvp_hbm.at[pl.ds(wid * NPAIR_W, NPAIR_W)], uvp_v)
    pltpu.sync_copy(cidx_hbm.at[pl.ds(base * L, BW * L)],
                    cidx_v.at[pl.ds(0, BW * L)])

    for q in range(BW // 128):
        pltpu.async_copy(feat_hbm.at[nodes_v.at[pl.ds(q * 128, 128)]],
                         out_v.at[pl.ds(q * 128, 128)], sem0).wait()
    pltpu.sync_copy(out_v, selff_hbm.at[pl.ds(base, BW)])

    def start(u, pair):
        pltpu.make_async_copy(
            y_hbm.at[uvp_v.at[pair]], rows_v.at[u], sems[u]).start()

    def wait(u):
        pltpu.make_async_copy(
            y_hbm.at[uvp_v.at[0]], rows_v.at[u], sems[u]).wait()

    inv_l = _f32(1.0 / L)
    zero = jnp.zeros((16,), _f32)

    def compute(u, pair):
        for bb in range(PAIR):
            cbase = pair * PROWS + bb * L

            def body(i, accs):
                ci = cidx_v[pl.ds(cbase + i, 16)][0]
                row = bb * L + i
                new = []
                for j in range(4):
                    y = rows_v[u, row, pl.ds(j * 16, 16)]
                    c = c_v[ci, pl.ds(j * 16, 16)]
                    new.append(accs[j] + jnp.maximum(y + c, 0.0))
                return tuple(new)

            accs = lax.fori_loop(0, L, body, (zero, zero, zero, zero))
            lb = pair * PAIR + bb
            for j in range(4):
                out_v[lb, pl.ds(j * 16, 16)] = accs[j] * inv_l

    for u in range(NBUF):
        start(u, u)

    def outer(t, _):
        p = t * NBUF
        for u in range(NBUF):
            pair = p + u
            wait(u)
            compute(u, pair)

            @pl.when(pair + NBUF < NPAIR_W)
            def _():
                start(u, pair + NBUF)
        return 0

    lax.fori_loop(0, NPAIR_W // NBUF, outer, 0)
    pltpu.sync_copy(out_v, neigh_hbm.at[pl.ds(base, BW)])


def _stage2(y, uvp, cidx, nodes, feat_table, c):
    mesh = plsc.VectorSubcoreMesh(core_axis_name="c", subcore_axis_name="s")
    kern = functools.partial(
        pl.kernel,
        mesh=mesh,
        compiler_params=pltpu.CompilerParams(use_tc_tiling_on_sc=False),
        out_type=[
            jax.ShapeDtypeStruct((B, D), _f32),
            jax.ShapeDtypeStruct((B, D), _f32),
        ],
        scratch_types=[
            pltpu.VMEM((NPAIR_W, PPAD), jnp.int32),
            pltpu.VMEM((BW * L + 16,), jnp.int32),
            pltpu.VMEM((32, D), _f32),
            pltpu.VMEM((BW,), jnp.int32),
            pltpu.VMEM((BW, D), _f32),
            pltpu.VMEM((NBUF, PPAD, D), _f32),
            pltpu.SemaphoreType.DMA,
            pltpu.SemaphoreType.DMA,
        ],
    )(_stage2_body)
    return kern(y, uvp, cidx, nodes, feat_table, c)


def _stage3_body(sf_ref, ng_ref, w1t_ref, w1b_ref, b1_ref, o_ref):
    o_ref[...] = jnp.maximum(
        jnp.dot(sf_ref[...], w1t_ref[...], preferred_element_type=_f32)
        + jnp.dot(ng_ref[...], w1b_ref[...], preferred_element_type=_f32)
        + b1_ref[...], 0.0)


def _stage3(selff, neigh, w1t, w1b, b1):
    rows = 1024
    return pl.pallas_call(
        _stage3_body,
        grid=(B // rows,),
        in_specs=[
            pl.BlockSpec((rows, D), lambda i: (i, 0)),
            pl.BlockSpec((rows, D), lambda i: (i, 0)),
            pl.BlockSpec((D, D), lambda i: (0, 0)),
            pl.BlockSpec((D, D), lambda i: (0, 0)),
            pl.BlockSpec((1, D), lambda i: (0, 0)),
        ],
        out_specs=pl.BlockSpec((rows, D), lambda i: (i, 0)),
        out_shape=jax.ShapeDtypeStruct((B, D), _f32),
    )(selff, neigh, w1t, w1b, b1)


def kernel(nodes, history_uv, history_ra, history_re, feat_table, e_table,
           r_table, re_table, W_gv, b_gv, W1, b1):
    wt = W_gv[:D]
    wb = W_gv[D:]
    rr = (r_table[:, None, :] + re_table[None, :, :]).reshape(R * R, D)
    rr = jnp.pad(rr, ((0, 32 - R * R), (0, 0)))
    y, c = _stage1(e_table, wt, rr, wb, b_gv.reshape(1, D))

    uvp = jnp.pad(history_uv.astype(jnp.int32).reshape(B // PAIR, PROWS),
                  ((0, 0), (0, PPAD - PROWS)))
    cidx = (history_ra * R + history_re).astype(jnp.int32).reshape(-1)
    neigh, selff = _stage2(y, uvp, cidx, nodes.astype(jnp.int32),
                           feat_table, c)

    return _stage3(selff, neigh, W1[:D], W1[D:], b1.reshape(1, D))
